# Initial kernel scaffold; baseline (speedup 1.0000x reference)
import jax
import jax.numpy as jnp
from jax.experimental import pallas as pl


def kernel(user_emb, item_emb, edge_vals, edge_index, users, items):
    raise NotImplementedError("write your pallas kernel here")


# speedup vs baseline: 1.0012x; 1.0012x over previous
import jax
import jax.numpy as jnp
from jax.experimental import pallas as pl

N_USER = 5000
M_ITEM = 5000
DIM = 128
LAYERS = 3
N_TOTAL = N_USER + M_ITEM


def _final_stage(u_ref, i_ref, ratings_ref, inter_ref, il_ref):
    u = u_ref[...]
    i = i_ref[...]
    il = u * i
    il_ref[...] = il
    um = jnp.mean(u, axis=0)
    im = jnp.mean(i, axis=0)
    inter = um * im
    inter_ref[...] = inter
    ratings_ref[...] = jax.nn.sigmoid(jnp.sum(inter, axis=1, keepdims=True))


def kernel(user_emb, item_emb, edge_vals, edge_index, users, items):
    all_emb = jnp.concatenate([user_emb, item_emb], axis=0)
    embs = [all_emb]
    src = edge_index[0]
    dst = edge_index[1]
    for _ in range(LAYERS):
        msg = edge_vals[:, None] * jnp.take(all_emb, src, axis=0)
        all_emb = jax.ops.segment_sum(msg, dst, num_segments=N_TOTAL)
        embs.append(all_emb)

    B = users.shape[0]
    u_stack = jnp.stack([jnp.take(e[:N_USER], users, axis=0) for e in embs])
    i_stack = jnp.stack([jnp.take(e[N_USER:], items, axis=0) for e in embs])

    ratings2d, inter, inter_layers = pl.pallas_call(
        _final_stage,
        out_shape=(
            jax.ShapeDtypeStruct((B, 1), jnp.float32),
            jax.ShapeDtypeStruct((B, DIM), jnp.float32),
            jax.ShapeDtypeStruct((LAYERS + 1, B, DIM), jnp.float32),
        ),
    )(u_stack, i_stack)
    return (ratings2d.reshape(B), inter, inter_layers)


# speedup vs baseline: 6.3671x; 6.3591x over previous
import functools

import jax
import jax.numpy as jnp
from jax import lax
from jax.experimental import pallas as pl
from jax.experimental.pallas import tpu as pltpu
from jax.experimental.pallas import tpu_sc as plsc

N_USER = 5000
M_ITEM = 5000
DIM = 128
LAYERS = 3
E = 320000
N_TOTAL = N_USER + M_ITEM

W = 128
N_WORKERS = 32
N_SUBCORES = 16
E_PAD = ((E + W * N_WORKERS - 1) // (W * N_WORKERS)) * (W * N_WORKERS)
N_WIN = E_PAD // W
N_PAD = 10240
ROWS_PER_SUBCORE = N_PAD // N_SUBCORES

_vector_mesh = plsc.VectorSubcoreMesh(core_axis_name="c", subcore_axis_name="s")


def _sc_layer(table, src2d, dst2d, vals2d, zeros_tab):
    @pl.kernel(
        out_type=jax.ShapeDtypeStruct((2, N_PAD, DIM), jnp.float32),
        mesh=_vector_mesh,
        scratch_types=[
            pltpu.VMEM((W, DIM), jnp.float32),
            pltpu.VMEM_SHARED((N_PAD, DIM), jnp.float32),
        ],
    )
    def layer_kernel(table_hbm, src_hbm, dst_hbm, vals_hbm, zeros_hbm,
                     out_hbm, rows_ref, acc_ref):
        cid = lax.axis_index("c")
        sid = lax.axis_index("s")
        row0 = sid * ROWS_PER_SUBCORE
        rows_slc = pl.ds(row0, ROWS_PER_SUBCORE)
        pltpu.sync_copy(zeros_hbm.at[rows_slc], acc_ref.at[rows_slc])
        plsc.subcore_barrier()

        def win_body(src_blk, dst_blk, vals_blk):
            pltpu.sync_copy(table_hbm.at[src_blk.at[0]], rows_ref)

            @pl.loop(0, W, step=16)
            def _(r16):
                vv = vals_blk[0, pl.ds(r16, 16)]
                for j in range(16):
                    v = vv[j]
                    for c in range(DIM // 16):
                        slc = (r16 + j, pl.ds(c * 16, 16))
                        rows_ref[slc] = rows_ref[slc] * v

            pltpu.sync_copy(rows_ref, acc_ref.at[dst_blk.at[0]], add=True)

        pltpu.emit_pipeline(
            win_body,
            grid=(N_WIN,),
            in_specs=[
                pl.BlockSpec((1, W), lambda i: (0, i)),
                pl.BlockSpec((1, W), lambda i: (0, i)),
                pl.BlockSpec((1, W), lambda i: (0, i)),
            ],
            out_specs=[],
            core_axis_name=("c", "s"),
            dimension_semantics=(pltpu.PARALLEL,),
        )(src_hbm, dst_hbm, vals_hbm)

        plsc.subcore_barrier()
        pltpu.sync_copy(acc_ref.at[rows_slc], out_hbm.at[cid, rows_slc])

    return layer_kernel(table, src2d, dst2d, vals2d, zeros_tab)


def _sc_gather(stacked_tables, gidx2d, n_out):
    @pl.kernel(
        out_type=jax.ShapeDtypeStruct((n_out, DIM), jnp.float32),
        mesh=_vector_mesh,
    )
    def gather_kernel(tab_hbm, idx_hbm, out_hbm):
        def body(idx_blk, out_blk):
            pltpu.sync_copy(tab_hbm.at[idx_blk.at[0]], out_blk)

        pltpu.emit_pipeline(
            body,
            grid=(n_out // W,),
            in_specs=[pl.BlockSpec((1, W), lambda i: (0, i))],
            out_specs=[pl.BlockSpec((W, DIM), lambda i: (i, 0))],
            core_axis_name=("c", "s"),
            dimension_semantics=(pltpu.PARALLEL,),
        )(idx_hbm, out_hbm)

    return gather_kernel(stacked_tables, gidx2d)


def _merge_body(p_ref, o_ref):
    o_ref[...] = p_ref[0, :N_TOTAL] + p_ref[1, :N_TOTAL]


def _merge(partials):
    return pl.pallas_call(
        _merge_body,
        out_shape=jax.ShapeDtypeStruct((N_TOTAL, DIM), jnp.float32),
    )(partials)


def _final_body(u_ref, i_ref, ratings_ref, inter_ref, il_ref):
    u = u_ref[...]
    i = i_ref[...]
    il = u * i
    il_ref[...] = il
    um = jnp.mean(u, axis=0)
    im = jnp.mean(i, axis=0)
    inter = um * im
    inter_ref[...] = inter
    ratings_ref[...] = jax.nn.sigmoid(jnp.sum(inter, axis=1, keepdims=True))


def kernel(user_emb, item_emb, edge_vals, edge_index, users, items):
    B = users.shape[0]
    table0 = jnp.concatenate([user_emb, item_emb], axis=0)

    pad = E_PAD - E
    pad_idx = jnp.arange(pad, dtype=jnp.int32) % N_TOTAL
    src2d = jnp.concatenate([edge_index[0], pad_idx]).reshape(1, E_PAD)
    dst2d = jnp.concatenate([edge_index[1], pad_idx]).reshape(1, E_PAD)
    vals2d = jnp.concatenate(
        [edge_vals, jnp.zeros((pad,), jnp.float32)]).reshape(1, E_PAD)
    zeros_tab = jnp.zeros((N_PAD, DIM), jnp.float32)

    tables = [table0]
    t = table0
    for _ in range(LAYERS):
        partials = _sc_layer(t, src2d, dst2d, vals2d, zeros_tab)
        t = _merge(partials)
        tables.append(t)

    stacked = jnp.concatenate(tables, axis=0)
    offs = jnp.arange(LAYERS + 1, dtype=jnp.int32)[:, None] * N_TOTAL
    gidx_u = offs + users[None, :]
    gidx_i = offs + N_USER + items[None, :]
    n_out = 2 * (LAYERS + 1) * B
    gidx2d = jnp.concatenate(
        [gidx_u.reshape(-1), gidx_i.reshape(-1)]).reshape(1, n_out)
    gathered = _sc_gather(stacked, gidx2d, n_out)
    u_stack = gathered[: (LAYERS + 1) * B].reshape(LAYERS + 1, B, DIM)
    i_stack = gathered[(LAYERS + 1) * B:].reshape(LAYERS + 1, B, DIM)

    ratings2d, inter, inter_layers = pl.pallas_call(
        _final_body,
        out_shape=(
            jax.ShapeDtypeStruct((B, 1), jnp.float32),
            jax.ShapeDtypeStruct((B, DIM), jnp.float32),
            jax.ShapeDtypeStruct((LAYERS + 1, B, DIM), jnp.float32),
        ),
    )(u_stack, i_stack)
    return (ratings2d.reshape(B), inter, inter_layers)


# speedup vs baseline: 10.4332x; 1.6386x over previous
import dataclasses
import functools

import jax
import jax.numpy as jnp
from jax import lax
from jax.experimental import pallas as pl
from jax.experimental.pallas import tpu as pltpu
from jax.experimental.pallas import tpu_sc as plsc

N_USER = 5000
M_ITEM = 5000
DIM = 128
LAYERS = 3
E = 320000
N_TOTAL = N_USER + M_ITEM

W = 128
N_WORKERS = 32
N_SUBCORES = 16
WPW = 81
E_PAD = W * N_WORKERS * WPW
N_WIN = E_PAD // W
ROW_CHUNK = 640

_vector_mesh = plsc.VectorSubcoreMesh(core_axis_name="c", subcore_axis_name="s")

_sc_params = pltpu.CompilerParams()
if "needs_layout_passes" in pltpu.CompilerParams.__dataclass_fields__:
    _sc_params = dataclasses.replace(_sc_params, needs_layout_passes=False)


def _sc_layer(table, idx3, zeros_tab):
    @pl.kernel(
        out_type=jax.ShapeDtypeStruct((2, N_TOTAL, DIM), jnp.float32),
        mesh=_vector_mesh,
        compiler_params=_sc_params,
        scratch_types=[
            pltpu.VMEM((3, W), jnp.int32),
            pltpu.VMEM((3, W), jnp.int32),
            pltpu.VMEM((3, W), jnp.int32),
            pltpu.VMEM((1, W), jnp.int32),
            pltpu.VMEM((1, W), jnp.int32),
            pltpu.VMEM((1, W), jnp.int32),
            pltpu.VMEM((W, DIM), jnp.float32),
            pltpu.VMEM((W, DIM), jnp.float32),
            pltpu.VMEM((W, DIM), jnp.float32),
            pltpu.VMEM_SHARED((N_TOTAL, DIM), jnp.float32),
            pltpu.SemaphoreType.DMA,
            pltpu.SemaphoreType.DMA,
            pltpu.SemaphoreType.DMA,
            pltpu.SemaphoreType.DMA,
            pltpu.SemaphoreType.DMA,
            pltpu.SemaphoreType.DMA,
            pltpu.SemaphoreType.DMA,
            pltpu.SemaphoreType.DMA,
            pltpu.SemaphoreType.DMA,
        ],
    )
    def layer_kernel(table_hbm, idx_hbm, zeros_hbm, out_hbm,
                     ix0, ix1, ix2, sx0, sx1, sx2, r0, r1, r2, acc_ref,
                     i0, i1, i2, g0, g1, g2, s0, s1, s2):
        idxs, scats, rows = [ix0, ix1, ix2], [sx0, sx1, sx2], [r0, r1, r2]
        isems, gsems, ssems = [i0, i1, i2], [g0, g1, g2], [s0, s1, s2]
        cid = lax.axis_index("c")
        sid = lax.axis_index("s")
        base = (cid * N_SUBCORES + sid) * WPW

        def start_idx(w, b):
            pltpu.async_copy(idx_hbm.at[base + w], idxs[b], isems[b])

        def wait_idx(b):
            pltpu.make_async_copy(idx_hbm.at[0], idxs[b], isems[b]).wait()

        def start_gather(w, b):
            pltpu.async_copy(table_hbm.at[idxs[b].at[0]], rows[b], gsems[b])

        def wait_gather(b):
            pltpu.make_async_copy(
                table_hbm.at[pl.ds(0, W)], rows[b], gsems[b]).wait()

        def start_scatter(b):
            pltpu.async_copy(
                rows[b], acc_ref.at[scats[b].at[0]], ssems[b], add=True)

        def wait_scatter(b):
            pltpu.make_async_copy(
                rows[b], acc_ref.at[pl.ds(0, W)], ssems[b]).wait()

        @pl.when(sid < N_SUBCORES - 1)
        def _():
            slc = pl.ds(sid * ROW_CHUNK, ROW_CHUNK)
            pltpu.sync_copy(zeros_hbm.at[slc], acc_ref.at[slc])

        @pl.when(sid == N_SUBCORES - 1)
        def _():
            slc = pl.ds((N_SUBCORES - 1) * ROW_CHUNK,
                        N_TOTAL - (N_SUBCORES - 1) * ROW_CHUNK)
            pltpu.sync_copy(zeros_hbm.at[slc], acc_ref.at[slc])

        plsc.subcore_barrier()

        for b in range(3):
            start_idx(b, b)
        for b in range(2):
            wait_idx(b)
            start_gather(b, b)

        @pl.loop(0, WPW, step=3)
        def _(g):
            for b in range(3):
                w = g + b
                wait_gather(b)
                for c in range(W // 16):
                    scats[b][0, pl.ds(c * 16, 16)] = \
                        idxs[b][1, pl.ds(c * 16, 16)]

                @pl.loop(0, W, step=16)
                def _(r16):
                    vv = plsc.bitcast(
                        idxs[b][2, pl.ds(r16, 16)], jnp.float32)
                    for j in range(16):
                        v = vv[j]
                        for c in range(DIM // 16):
                            slc = (r16 + j, pl.ds(c * 16, 16))
                            rows[b][slc] = rows[b][slc] * v

                start_scatter(b)

                @pl.when(w + 3 < WPW)
                def _():
                    start_idx(w + 3, b)

                bn = (b + 2) % 3

                @pl.when(w + 2 < WPW)
                def _():
                    @pl.when(w >= 1)
                    def _():
                        wait_scatter(bn)

                    wait_idx(bn)
                    start_gather(w + 2, bn)

        for b in range(3):
            wait_scatter(b)
        plsc.subcore_barrier()

        @pl.when(sid < N_SUBCORES - 1)
        def _():
            slc = pl.ds(sid * ROW_CHUNK, ROW_CHUNK)
            pltpu.sync_copy(acc_ref.at[slc], out_hbm.at[cid, slc])

        @pl.when(sid == N_SUBCORES - 1)
        def _():
            slc = pl.ds((N_SUBCORES - 1) * ROW_CHUNK,
                        N_TOTAL - (N_SUBCORES - 1) * ROW_CHUNK)
            pltpu.sync_copy(acc_ref.at[slc], out_hbm.at[cid, slc])

    return layer_kernel(table, idx3, zeros_tab)


def _sc_gather(stacked_tables, gidx2d, n_out):
    @pl.kernel(
        out_type=jax.ShapeDtypeStruct((n_out, DIM), jnp.float32),
        mesh=_vector_mesh,
    )
    def gather_kernel(tab_hbm, idx_hbm, out_hbm):
        def body(idx_blk, out_blk):
            pltpu.sync_copy(tab_hbm.at[idx_blk.at[0]], out_blk)

        pltpu.emit_pipeline(
            body,
            grid=(n_out // W,),
            in_specs=[pl.BlockSpec((1, W), lambda i: (0, i))],
            out_specs=[pl.BlockSpec((W, DIM), lambda i: (i, 0))],
            core_axis_name=("c", "s"),
            dimension_semantics=(pltpu.PARALLEL,),
        )(idx_hbm, out_hbm)

    return gather_kernel(stacked_tables, gidx2d)


def _merge_body(p_ref, o_ref):
    o_ref[...] = p_ref[0] + p_ref[1]


def _merge(partials):
    return pl.pallas_call(
        _merge_body,
        out_shape=jax.ShapeDtypeStruct((N_TOTAL, DIM), jnp.float32),
    )(partials)


def _final_body(u_ref, i_ref, ratings_ref, inter_ref, il_ref):
    u = u_ref[...]
    i = i_ref[...]
    il = u * i
    il_ref[...] = il
    um = jnp.mean(u, axis=0)
    im = jnp.mean(i, axis=0)
    inter = um * im
    inter_ref[...] = inter
    ratings_ref[...] = jax.nn.sigmoid(jnp.sum(inter, axis=1, keepdims=True))


def kernel(user_emb, item_emb, edge_vals, edge_index, users, items):
    B = users.shape[0]
    table0 = jnp.concatenate([user_emb, item_emb], axis=0)

    pad = E_PAD - E
    pad_idx = jnp.arange(pad, dtype=jnp.int32) % N_TOTAL
    src_w = jnp.concatenate([edge_index[0], pad_idx]).reshape(N_WIN, 1, W)
    dst_w = jnp.concatenate([edge_index[1], pad_idx]).reshape(N_WIN, 1, W)
    vals_w = lax.bitcast_convert_type(
        jnp.concatenate([edge_vals, jnp.zeros((pad,), jnp.float32)]),
        jnp.int32).reshape(N_WIN, 1, W)
    idx3 = jnp.concatenate([src_w, dst_w, vals_w], axis=1)
    zeros_tab = jnp.zeros((N_TOTAL, DIM), jnp.float32)

    tables = [table0]
    t = table0
    for _ in range(LAYERS):
        partials = _sc_layer(t, idx3, zeros_tab)
        t = _merge(partials)
        tables.append(t)

    stacked = jnp.concatenate(tables, axis=0)
    offs = jnp.arange(LAYERS + 1, dtype=jnp.int32)[:, None] * N_TOTAL
    gidx_u = offs + users[None, :]
    gidx_i = offs + N_USER + items[None, :]
    n_out = 2 * (LAYERS + 1) * B
    gidx2d = jnp.concatenate(
        [gidx_u.reshape(-1), gidx_i.reshape(-1)]).reshape(1, n_out)
    gathered = _sc_gather(stacked, gidx2d, n_out)
    u_stack = gathered[: (LAYERS + 1) * B].reshape(LAYERS + 1, B, DIM)
    i_stack = gathered[(LAYERS + 1) * B:].reshape(LAYERS + 1, B, DIM)

    ratings2d, inter, inter_layers = pl.pallas_call(
        _final_body,
        out_shape=(
            jax.ShapeDtypeStruct((B, 1), jnp.float32),
            jax.ShapeDtypeStruct((B, DIM), jnp.float32),
            jax.ShapeDtypeStruct((LAYERS + 1, B, DIM), jnp.float32),
        ),
    )(u_stack, i_stack)
    return (ratings2d.reshape(B), inter, inter_layers)
